# Initial kernel scaffold; baseline (speedup 1.0000x reference)
import jax
import jax.numpy as jnp
from jax.experimental import pallas as pl


def kernel(x, a, V0, C0, S0, b0, V1, C1, S1, b1, V2, C2, S2, b2, V3, C3, S3, b3, Wd1, bd1, Wd2, bd2):
    raise NotImplementedError("write your pallas kernel here")


# speedup vs baseline: 1.7778x; 1.7778x over previous
import jax
import jax.numpy as jnp
from jax.experimental import pallas as pl
from jax.experimental.pallas import tpu as pltpu

N = 2048
BLK = 256
NBLK = N // BLK
NLAYER = 4
NRATE = 10


def _body(a_ref, x_ref, V0_ref, Vs_ref, C0_ref, Cs_ref, S0_ref, Ss_ref,
          b0_ref, bs_ref, Wd1_ref, bd1_ref, Wd2_ref, bd2_ref,
          out_ref, states_ref, hw_ref):
    l = pl.program_id(0)
    ib = pl.program_id(1)
    r0 = ib * BLK

    @pl.when(jnp.logical_and(l == 0, ib == 0))
    def _():
        xf = x_ref[...]
        hw_ref[:, 0:32] = jnp.dot(xf, V0_ref[0], preferred_element_type=jnp.float32)
        hw_ref[:, 32:64] = jnp.dot(xf, V0_ref[1], preferred_element_type=jnp.float32)

    @pl.when(jnp.logical_and(l > 0, ib == 0))
    def _():
        li = l - 1
        hprev = states_ref[li]
        hw_ref[:, 0:32] = jnp.dot(hprev, Vs_ref[li, 0], preferred_element_type=jnp.float32)
        hw_ref[:, 32:64] = jnp.dot(hprev, Vs_ref[li, 1], preferred_element_type=jnp.float32)

    ablk = a_ref[...]

    def block_msg(c0s, c1s):
        A0 = jnp.zeros((BLK, N), jnp.float32)
        A1 = jnp.zeros((BLK, N), jnp.float32)
        for r in range(NRATE):
            m = (ablk == jnp.float32(r + 1)).astype(jnp.float32)
            cnt = jnp.sum(m, axis=1, keepdims=True)
            s = 1.0 / (cnt + 1.0)
            A0 = A0 + m * (s * c0s[r])
            A1 = A1 + m * (s * c1s[r])
        msg = jnp.dot(A0, hw_ref[:, 0:32], preferred_element_type=jnp.float32)
        msg = msg + jnp.dot(A1, hw_ref[:, 32:64], preferred_element_type=jnp.float32)
        return msg

    @pl.when(l == 0)
    def _():
        c0s = [C0_ref[r, 0] for r in range(NRATE)]
        c1s = [C0_ref[r, 1] for r in range(NRATE)]
        msg = block_msg(c0s, c1s)
        hin = x_ref[pl.ds(r0, BLK), :]
        selfterm = jnp.dot(hin, S0_ref[...], preferred_element_type=jnp.float32) + b0_ref[...]
        states_ref[0, pl.ds(r0, BLK), :] = jnp.tanh(selfterm + msg)

    @pl.when(l > 0)
    def _():
        li = l - 1
        c0s = [Cs_ref[li, r, 0] for r in range(NRATE)]
        c1s = [Cs_ref[li, r, 1] for r in range(NRATE)]
        msg = block_msg(c0s, c1s)
        hin = states_ref[li, pl.ds(r0, BLK), :]
        selfterm = jnp.dot(hin, Ss_ref[li], preferred_element_type=jnp.float32) + bs_ref[li]
        states_ref[l, pl.ds(r0, BLK), :] = jnp.tanh(selfterm + msg)

    @pl.when(jnp.logical_and(l == NLAYER - 1, ib == NBLK - 1))
    def _():
        parts = [states_ref[k, 0:64, :] for k in range(NLAYER)]
        parts += [states_ref[k, 64:128, :] for k in range(NLAYER)]
        feat = jnp.concatenate(parts, axis=1)
        hdn = jnp.dot(feat, Wd1_ref[...], preferred_element_type=jnp.float32) + bd1_ref[...]
        hdn = jnp.maximum(hdn, 0.0)
        out_ref[...] = jnp.dot(hdn, Wd2_ref[...], preferred_element_type=jnp.float32) + bd2_ref[...]


def kernel(x, a, V0, C0, S0, b0, V1, C1, S1, b1, V2, C2, S2, b2, V3, C3, S3, b3, Wd1, bd1, Wd2, bd2):
    Vs = jnp.stack([V1, V2, V3])
    Cs = jnp.stack([C1, C2, C3])
    Ss = jnp.stack([S1, S2, S3])
    bs = jnp.stack([b1, b2, b3]).reshape(3, 1, 32)
    b0r = b0.reshape(1, 32)
    bd1r = bd1.reshape(1, 128)
    bd2r = bd2.reshape(1, 1)

    full = lambda shape: pl.BlockSpec(shape, lambda l, ib: (0,) * len(shape))
    smem = pl.BlockSpec(memory_space=pltpu.SMEM)

    out = pl.pallas_call(
        _body,
        grid=(NLAYER, NBLK),
        in_specs=[
            pl.BlockSpec((BLK, N), lambda l, ib: (ib, 0)),
            full((N, 128)),
            full((2, 128, 32)),
            full((3, 2, 32, 32)),
            smem,
            smem,
            full((128, 32)),
            full((3, 32, 32)),
            full((1, 32)),
            full((3, 1, 32)),
            full((2 * 128, 128)),
            full((1, 128)),
            full((128, 1)),
            full((1, 1)),
        ],
        out_specs=pl.BlockSpec((64, 1), lambda l, ib: (0, 0)),
        out_shape=jax.ShapeDtypeStruct((64, 1), jnp.float32),
        scratch_shapes=[
            pltpu.VMEM((NLAYER, N, 32), jnp.float32),
            pltpu.VMEM((N, 64), jnp.float32),
        ],
    )(a, x, V0, Vs, C0, Cs, S0, Ss, b0r, bs, Wd1, bd1r, Wd2, bd2r)
    return out


# speedup vs baseline: 2.3302x; 1.3107x over previous
import jax
import jax.numpy as jnp
from jax.experimental import pallas as pl
from jax.experimental.pallas import tpu as pltpu

N = 2048
BLK = 256
NBLK = N // BLK
NLAYER = 4
NRATE = 10


def _body(a_ref, x_ref, V0_ref, Vs_ref, C0_ref, Cs_ref, S0_ref, Ss_ref,
          b0_ref, bs_ref, Wd1_ref, bd1_ref, Wd2_ref, bd2_ref,
          out_ref, states_ref, hw_ref, stab_ref):
    l = pl.program_id(0)
    ib = pl.program_id(1)
    r0 = ib * BLK

    @pl.when(jnp.logical_and(l == 0, ib == 0))
    def _():
        xf = x_ref[...]
        hw_ref[:, 0:32] = jnp.dot(xf, V0_ref[0], preferred_element_type=jnp.float32)
        hw_ref[:, 32:64] = jnp.dot(xf, V0_ref[1], preferred_element_type=jnp.float32)

    @pl.when(jnp.logical_and(l > 0, ib == 0))
    def _():
        li = l - 1
        hprev = states_ref[li]
        hw_ref[:, 0:32] = jnp.dot(hprev, Vs_ref[li, 0], preferred_element_type=jnp.float32)
        hw_ref[:, 32:64] = jnp.dot(hprev, Vs_ref[li, 1], preferred_element_type=jnp.float32)

    ablk = a_ref[...]

    @pl.when(l == 0)
    def _():
        for r in range(NRATE):
            m = (ablk == jnp.float32(r + 1)).astype(jnp.float32)
            cnt = jnp.sum(m, axis=1, keepdims=True)
            stab_ref[pl.ds(r0, BLK), r:r + 1] = 1.0 / (cnt + 1.0)

    def block_msg(c0s, c1s):
        t0 = [stab_ref[pl.ds(r0, BLK), r:r + 1] * c0s[r] for r in range(NRATE)]
        t1 = [stab_ref[pl.ds(r0, BLK), r:r + 1] * c1s[r] for r in range(NRATE)]
        A0 = jnp.zeros((BLK, N), jnp.float32)
        A1 = jnp.zeros((BLK, N), jnp.float32)
        for r in range(NRATE):
            m = ablk == jnp.float32(r + 1)
            A0 = jnp.where(m, t0[r], A0)
            A1 = jnp.where(m, t1[r], A1)
        msg = jnp.dot(A0, hw_ref[:, 0:32], preferred_element_type=jnp.float32)
        msg = msg + jnp.dot(A1, hw_ref[:, 32:64], preferred_element_type=jnp.float32)
        return msg

    @pl.when(l == 0)
    def _():
        c0s = [C0_ref[r, 0] for r in range(NRATE)]
        c1s = [C0_ref[r, 1] for r in range(NRATE)]
        msg = block_msg(c0s, c1s)
        hin = x_ref[pl.ds(r0, BLK), :]
        selfterm = jnp.dot(hin, S0_ref[...], preferred_element_type=jnp.float32) + b0_ref[...]
        states_ref[0, pl.ds(r0, BLK), :] = jnp.tanh(selfterm + msg)

    @pl.when(l > 0)
    def _():
        li = l - 1
        c0s = [Cs_ref[li, r, 0] for r in range(NRATE)]
        c1s = [Cs_ref[li, r, 1] for r in range(NRATE)]
        msg = block_msg(c0s, c1s)
        hin = states_ref[li, pl.ds(r0, BLK), :]
        selfterm = jnp.dot(hin, Ss_ref[li], preferred_element_type=jnp.float32) + bs_ref[li]
        states_ref[l, pl.ds(r0, BLK), :] = jnp.tanh(selfterm + msg)

    @pl.when(jnp.logical_and(l == NLAYER - 1, ib == NBLK - 1))
    def _():
        parts = [states_ref[k, 0:64, :] for k in range(NLAYER)]
        parts += [states_ref[k, 64:128, :] for k in range(NLAYER)]
        feat = jnp.concatenate(parts, axis=1)
        hdn = jnp.dot(feat, Wd1_ref[...], preferred_element_type=jnp.float32) + bd1_ref[...]
        hdn = jnp.maximum(hdn, 0.0)
        out_ref[...] = jnp.dot(hdn, Wd2_ref[...], preferred_element_type=jnp.float32) + bd2_ref[...]


def kernel(x, a, V0, C0, S0, b0, V1, C1, S1, b1, V2, C2, S2, b2, V3, C3, S3, b3, Wd1, bd1, Wd2, bd2):
    Vs = jnp.stack([V1, V2, V3])
    Cs = jnp.stack([C1, C2, C3])
    Ss = jnp.stack([S1, S2, S3])
    bs = jnp.stack([b1, b2, b3]).reshape(3, 1, 32)
    b0r = b0.reshape(1, 32)
    bd1r = bd1.reshape(1, 128)
    bd2r = bd2.reshape(1, 1)

    full = lambda shape: pl.BlockSpec(shape, lambda l, ib: (0,) * len(shape))
    smem = pl.BlockSpec(memory_space=pltpu.SMEM)

    out = pl.pallas_call(
        _body,
        grid=(NLAYER, NBLK),
        in_specs=[
            pl.BlockSpec((BLK, N), lambda l, ib: (ib, 0)),
            full((N, 128)),
            full((2, 128, 32)),
            full((3, 2, 32, 32)),
            smem,
            smem,
            full((128, 32)),
            full((3, 32, 32)),
            full((1, 32)),
            full((3, 1, 32)),
            full((2 * 128, 128)),
            full((1, 128)),
            full((128, 1)),
            full((1, 1)),
        ],
        out_specs=pl.BlockSpec((64, 1), lambda l, ib: (0, 0)),
        out_shape=jax.ShapeDtypeStruct((64, 1), jnp.float32),
        scratch_shapes=[
            pltpu.VMEM((NLAYER, N, 32), jnp.float32),
            pltpu.VMEM((N, 64), jnp.float32),
            pltpu.VMEM((N, 16), jnp.float32),
        ],
    )(a, x, V0, Vs, C0, Cs, S0, Ss, b0r, bs, Wd1, bd1r, Wd2, bd2r)
    return out
